# baseline (speedup 1.0000x reference)
import functools

import jax
import jax.numpy as jnp
import numpy as np
from jax import lax
from jax.experimental import pallas as pl
from jax.experimental.pallas import tpu as pltpu
from jax.experimental.pallas import tpu_sc as plsc

N = 10000
E = 320000
D = 128
H = 128
NP = 10240
EOS = 1e-10
ALPHA = 1.0
TEMP = 1.0
BIAS = 0.0001
NC = 2
NS = 16
NW = NC * NS
CPT = E // NW
L = 16
TAIL_TILES = 25
TAIL_CPT = N // TAIL_TILES

def _gumbel_logit():
    def threefry2x32(k1, k2, x0, x1):
        rotations = [np.array([13, 15, 26, 6]), np.array([17, 29, 16, 24])]
        ks = [np.uint32(k1), np.uint32(k2),
              np.uint32(k1) ^ np.uint32(k2) ^ np.uint32(0x1BD11BDA)]
        x = [(x0 + ks[0]).astype(np.uint32), (x1 + ks[1]).astype(np.uint32)]
        for i in range(5):
            for r in rotations[i % 2]:
                x[0] = (x[0] + x[1]).astype(np.uint32)
                x[1] = ((x[1] << np.uint32(r))
                        | (x[1] >> np.uint32(32 - r))).astype(np.uint32)
                x[1] = x[1] ^ x[0]
            x[0] = (x[0] + ks[(i + 1) % 3]).astype(np.uint32)
            x[1] = (x[1] + ks[(i + 2) % 3] + np.uint32(i + 1)).astype(np.uint32)
        return x

    b1, b2 = threefry2x32(0, 42, np.zeros((E,), np.uint32),
                          np.arange(E, dtype=np.uint32))
    bits = b1 ^ b2
    u = ((bits >> np.uint32(9)) | np.uint32(0x3F800000)).view(np.float32) \
        - np.float32(1.0)
    u = np.maximum(np.float32(0.0), u)
    eps = (np.float32(BIAS - (1.0 - BIAS)) * u + np.float32(1.0 - BIAS))
    return np.log(eps) - np.log(np.float32(1.0) - eps)


_G_CONST = _gumbel_logit()


K1_BLK = 128
K1_GRID = (N + K1_BLK - 1) // K1_BLK


def _k1_body(x_ref, w_ref, b_ref, v_ref, be_ref, p_ref):
    h = jax.nn.relu(
        jnp.dot(x_ref[...], w_ref[...], preferred_element_type=jnp.float32)
        + b_ref[...])
    p = lax.dot_general(
        v_ref[...], h, (((1,), (1,)), ((), ())),
        preferred_element_type=jnp.float32) + be_ref[...]
    p_ref[...] = p.reshape(1, 1, K1_BLK)


def _run_k1(x, w_emb, b_emb2, v2, be2):
    return pl.pallas_call(
        _k1_body,
        grid=(K1_GRID,),
        in_specs=[
            pl.BlockSpec((K1_BLK, D), lambda a: (a, 0)),
            pl.BlockSpec((D, H), lambda a: (0, 0)),
            pl.BlockSpec((1, H), lambda a: (0, 0)),
            pl.BlockSpec((1, H), lambda a: (0, 0)),
            pl.BlockSpec((1, 1), lambda a: (0, 0)),
        ],
        out_specs=pl.BlockSpec((1, 1, K1_BLK), lambda a: (a, 0, 0)),
        out_shape=jax.ShapeDtypeStruct((K1_GRID, 1, K1_BLK), jnp.float32),
    )(x, w_emb, b_emb2, v2, be2)


_MESH = plsc.VectorSubcoreMesh(
    core_axis_name="c", subcore_axis_name="s", num_cores=NC, num_subcores=NS)
_SC_PARAMS = pltpu.CompilerParams(
    needs_layout_passes=False, use_tc_tiling_on_sc=False)


@functools.partial(
    pl.kernel,
    out_type=[
        jax.ShapeDtypeStruct((E,), jnp.float32),
        jax.ShapeDtypeStruct((E,), jnp.float32),
        jax.ShapeDtypeStruct((NW, NP), jnp.float32),
        jax.ShapeDtypeStruct((NW, NP), jnp.float32),
    ],
    mesh=_MESH,
    compiler_params=_SC_PARAMS,
    scratch_types=[
        pltpu.VMEM((N,), jnp.float32),
        pltpu.VMEM((CPT,), jnp.int32),
        pltpu.VMEM((CPT,), jnp.int32),
        pltpu.VMEM((CPT,), jnp.float32),
        pltpu.VMEM((CPT,), jnp.float32),
        pltpu.VMEM((CPT,), jnp.float32),
        pltpu.VMEM((NP,), jnp.float32),
        pltpu.VMEM((NP,), jnp.float32),
    ],
)
def _k2(p_hbm, edges_hbm, g_hbm,
        wlp_hbm, whp_hbm, dlp_hbm, dhp_hbm,
        p_v, src_v, dst_v, g_v, wlp_v, whp_v, dlp_v, dhp_v):
    wid = lax.axis_index("s") * NC + lax.axis_index("c")
    base = wid * CPT
    pltpu.sync_copy(p_hbm.at[pl.ds(0, N)], p_v)
    pltpu.sync_copy(edges_hbm.at[0, pl.ds(base, CPT)], src_v)
    pltpu.sync_copy(edges_hbm.at[1, pl.ds(base, CPT)], dst_v)
    pltpu.sync_copy(g_hbm.at[pl.ds(base, CPT)], g_v)

    zero = jnp.zeros((L,), jnp.float32)

    @plsc.parallel_loop(0, NP // L, 1, unroll=8)
    def _zero(i):
        dlp_v[pl.ds(i * L, L)] = zero
        dhp_v[pl.ds(i * L, L)] = zero

    @plsc.parallel_loop(0, CPT // L, 1, unroll=5)
    def _edge(i):
        o = i * L
        s = src_v[pl.ds(o, L)]
        d = dst_v[pl.ds(o, L)]
        gg = g_v[pl.ds(o, L)]
        ps = plsc.load_gather(p_v, [s])
        pd = plsc.load_gather(p_v, [d])
        gate = gg + ps + pd
        wlp = 1.0 / (1.0 + jnp.exp(-gate))
        whp = 1.0 - wlp
        wlp_v[pl.ds(o, L)] = wlp
        whp_v[pl.ds(o, L)] = whp
        plsc.addupdate_scatter(dlp_v, [s], wlp + EOS)
        plsc.addupdate_scatter(dhp_v, [s], whp + EOS)

    pltpu.sync_copy(wlp_v, wlp_hbm.at[pl.ds(base, CPT)])
    pltpu.sync_copy(whp_v, whp_hbm.at[pl.ds(base, CPT)])
    pltpu.sync_copy(dlp_v, dlp_hbm.at[wid])
    pltpu.sync_copy(dhp_v, dhp_hbm.at[wid])


def _k3_body(dlp_ref, dhp_ref, dislp_ref, dishp_ref):
    deg_lp = jnp.sum(dlp_ref[...], axis=0, keepdims=True) + (1.0 + EOS)
    deg_hp = jnp.sum(dhp_ref[...], axis=0, keepdims=True) + (1.0 + EOS)
    dislp_ref[...] = lax.rsqrt(deg_lp + EOS)
    dishp_ref[...] = lax.rsqrt(deg_hp + EOS)


def _run_k3(dlp_part, dhp_part):
    return pl.pallas_call(
        _k3_body,
        out_shape=[
            jax.ShapeDtypeStruct((1, NP), jnp.float32),
            jax.ShapeDtypeStruct((1, NP), jnp.float32),
        ],
    )(dlp_part, dhp_part)


@functools.partial(
    pl.kernel,
    out_type=[
        jax.ShapeDtypeStruct((E + N,), jnp.float32),
        jax.ShapeDtypeStruct((E + N,), jnp.float32),
    ],
    mesh=_MESH,
    compiler_params=_SC_PARAMS,
    scratch_types=[
        pltpu.VMEM((N,), jnp.float32),
        pltpu.VMEM((N,), jnp.float32),
        pltpu.VMEM((CPT,), jnp.int32),
        pltpu.VMEM((CPT,), jnp.int32),
        pltpu.VMEM((CPT,), jnp.float32),
        pltpu.VMEM((CPT,), jnp.float32),
        pltpu.VMEM((CPT,), jnp.float32),
        pltpu.VMEM((TAIL_CPT,), jnp.float32),
        pltpu.VMEM((TAIL_CPT,), jnp.float32),
    ],
)
def _k4(dislp_hbm, dishp_hbm, edges_hbm, wlp_hbm,
        olp_hbm, ohp_hbm,
        dislp_v, dishp_v, src_v, dst_v, wlp_v, olp_v, ohp_v, tlp_v, thp_v):
    wid = lax.axis_index("s") * NC + lax.axis_index("c")
    base = wid * CPT
    pltpu.sync_copy(dislp_hbm.at[pl.ds(0, N)], dislp_v)
    pltpu.sync_copy(dishp_hbm.at[pl.ds(0, N)], dishp_v)
    pltpu.sync_copy(edges_hbm.at[0, pl.ds(base, CPT)], src_v)
    pltpu.sync_copy(edges_hbm.at[1, pl.ds(base, CPT)], dst_v)
    pltpu.sync_copy(wlp_hbm.at[pl.ds(base, CPT)], wlp_v)

    @plsc.parallel_loop(0, CPT // L, 1, unroll=5)
    def _edge(i):
        o = i * L
        s = src_v[pl.ds(o, L)]
        d = dst_v[pl.ds(o, L)]
        w = wlp_v[pl.ds(o, L)]
        a = plsc.load_gather(dislp_v, [s]) * plsc.load_gather(dislp_v, [d])
        b = plsc.load_gather(dishp_v, [s]) * plsc.load_gather(dishp_v, [d])
        olp_v[pl.ds(o, L)] = (w + EOS) * a
        ohp_v[pl.ds(o, L)] = -((1.0 - w) + EOS) * b

    pltpu.sync_copy(olp_v, olp_hbm.at[pl.ds(base, CPT)])
    pltpu.sync_copy(ohp_v, ohp_hbm.at[pl.ds(base, CPT)])

    @pl.when(wid < TAIL_TILES)
    def _tail():
        tb = wid * TAIL_CPT
        ones = jnp.ones((L,), jnp.float32)

        @plsc.parallel_loop(0, TAIL_CPT // L, 1, unroll=5)
        def _t(k):
            dl = dislp_v[pl.ds(tb + k * L, L)]
            tlp_v[pl.ds(k * L, L)] = (1.0 + EOS) * dl * dl
            thp_v[pl.ds(k * L, L)] = ones

        pltpu.sync_copy(tlp_v, olp_hbm.at[pl.ds(E + tb, TAIL_CPT)])
        pltpu.sync_copy(thp_v, ohp_hbm.at[pl.ds(E + tb, TAIL_CPT)])


def kernel(features, edges, W_emb, b_emb, W_edge, b_edge):
    b_emb2 = b_emb.reshape(1, H)
    v2 = (0.5 * (W_edge[:H, 0] + W_edge[H:, 0])).reshape(1, H)
    be2 = (0.5 * b_edge).reshape(1, 1)
    g = jnp.asarray(_G_CONST)

    p = _run_k1(features, W_emb, b_emb2, v2, be2).reshape(K1_GRID * K1_BLK)

    wlp, whp, dlp_part, dhp_part = _k2(p, edges, g)

    dis_lp, dis_hp = _run_k3(dlp_part, dhp_part)

    w_lp_norm, w_hp_norm = _k4(
        dis_lp.reshape(NP), dis_hp.reshape(NP), edges, wlp)

    loop = jnp.arange(N, dtype=edges.dtype)
    edges_full = jnp.concatenate([edges, jnp.stack([loop, loop])], axis=1)
    return (edges_full, w_lp_norm, edges_full, w_hp_norm, wlp, whp)


# speedup vs baseline: 1.8203x; 1.8203x over previous
import functools

import jax
import jax.numpy as jnp
import numpy as np
from jax import lax
from jax.experimental import pallas as pl
from jax.experimental.pallas import tpu as pltpu
from jax.experimental.pallas import tpu_sc as plsc

N = 10000
E = 320000
D = 128
H = 128
NP = 10240
EOS = 1e-10
ALPHA = 1.0
TEMP = 1.0
BIAS = 0.0001
NC = 2
NS = 16
NW = NC * NS
CPT = E // NW
L = 16
TAIL_TILES = 25
TAIL_CPT = N // TAIL_TILES


def _gumbel_logit():
    def threefry2x32(k1, k2, x0, x1):
        rotations = [np.array([13, 15, 26, 6]), np.array([17, 29, 16, 24])]
        ks = [np.uint32(k1), np.uint32(k2),
              np.uint32(k1) ^ np.uint32(k2) ^ np.uint32(0x1BD11BDA)]
        x = [(x0 + ks[0]).astype(np.uint32), (x1 + ks[1]).astype(np.uint32)]
        for i in range(5):
            for r in rotations[i % 2]:
                x[0] = (x[0] + x[1]).astype(np.uint32)
                x[1] = ((x[1] << np.uint32(r))
                        | (x[1] >> np.uint32(32 - r))).astype(np.uint32)
                x[1] = x[1] ^ x[0]
            x[0] = (x[0] + ks[(i + 1) % 3]).astype(np.uint32)
            x[1] = (x[1] + ks[(i + 2) % 3] + np.uint32(i + 1)).astype(np.uint32)
        return x

    b1, b2 = threefry2x32(0, 42, np.zeros((E,), np.uint32),
                          np.arange(E, dtype=np.uint32))
    bits = b1 ^ b2
    u = ((bits >> np.uint32(9)) | np.uint32(0x3F800000)).view(np.float32) \
        - np.float32(1.0)
    u = np.maximum(np.float32(0.0), u)
    eps = (np.float32(BIAS - (1.0 - BIAS)) * u + np.float32(1.0 - BIAS))
    return np.log(eps) - np.log(np.float32(1.0) - eps)


_G_CONST = _gumbel_logit()


def _k1_body(x_ref, w_ref, b_ref, v_ref, be_ref, e_ref,
             p_ref, src_ref, dst_ref):
    h = jax.nn.relu(
        jnp.dot(x_ref[...], w_ref[...], preferred_element_type=jnp.float32)
        + b_ref[...])
    p = lax.dot_general(
        v_ref[...], h, (((1,), (1,)), ((), ())),
        preferred_element_type=jnp.float32) + be_ref[...]
    p_ref[...] = p.reshape(N)
    e = e_ref[...]
    src_ref[...] = e[0]
    dst_ref[...] = e[1]


def _run_k1(x, w_emb, b_emb2, v2, be2, edges):
    return pl.pallas_call(
        _k1_body,
        out_shape=[
            jax.ShapeDtypeStruct((N,), jnp.float32),
            jax.ShapeDtypeStruct((E,), jnp.int32),
            jax.ShapeDtypeStruct((E,), jnp.int32),
        ],
    )(x, w_emb, b_emb2, v2, be2, edges)


_MESH = plsc.VectorSubcoreMesh(
    core_axis_name="c", subcore_axis_name="s", num_cores=NC, num_subcores=NS)
_SC_PARAMS = pltpu.CompilerParams(needs_layout_passes=False)


@functools.partial(
    pl.kernel,
    out_type=[
        jax.ShapeDtypeStruct((E,), jnp.float32),
        jax.ShapeDtypeStruct((E,), jnp.float32),
        jax.ShapeDtypeStruct((NW * NP,), jnp.float32),
        jax.ShapeDtypeStruct((NW * NP,), jnp.float32),
    ],
    mesh=_MESH,
    compiler_params=_SC_PARAMS,
    scratch_types=[
        pltpu.VMEM((N,), jnp.float32),
        pltpu.VMEM((CPT,), jnp.int32),
        pltpu.VMEM((CPT,), jnp.int32),
        pltpu.VMEM((CPT,), jnp.float32),
        pltpu.VMEM((CPT,), jnp.float32),
        pltpu.VMEM((CPT,), jnp.float32),
        pltpu.VMEM((NP,), jnp.float32),
        pltpu.VMEM((NP,), jnp.float32),
    ],
)
def _k2(p_hbm, src_hbm, dst_hbm, g_hbm,
        wlp_hbm, whp_hbm, dlp_hbm, dhp_hbm,
        p_v, src_v, dst_v, g_v, wlp_v, whp_v, dlp_v, dhp_v):
    wid = lax.axis_index("s") * NC + lax.axis_index("c")
    base = wid * CPT
    pltpu.sync_copy(p_hbm.at[pl.ds(0, N)], p_v)
    pltpu.sync_copy(src_hbm.at[pl.ds(base, CPT)], src_v)
    pltpu.sync_copy(dst_hbm.at[pl.ds(base, CPT)], dst_v)
    pltpu.sync_copy(g_hbm.at[pl.ds(base, CPT)], g_v)

    zero = jnp.zeros((L,), jnp.float32)

    @plsc.parallel_loop(0, NP // L, 1, unroll=8)
    def _zero(i):
        dlp_v[pl.ds(i * L, L)] = zero
        dhp_v[pl.ds(i * L, L)] = zero

    @plsc.parallel_loop(0, CPT // L, 1, unroll=5)
    def _edge(i):
        o = i * L
        s = src_v[pl.ds(o, L)]
        d = dst_v[pl.ds(o, L)]
        gg = g_v[pl.ds(o, L)]
        ps = plsc.load_gather(p_v, [s])
        pd = plsc.load_gather(p_v, [d])
        gate = gg + ps + pd
        wlp = 1.0 / (1.0 + jnp.exp(-gate))
        whp = 1.0 - wlp
        wlp_v[pl.ds(o, L)] = wlp
        whp_v[pl.ds(o, L)] = whp
        plsc.addupdate_scatter(dlp_v, [s], wlp + EOS)
        plsc.addupdate_scatter(dhp_v, [s], whp + EOS)

    pltpu.sync_copy(wlp_v, wlp_hbm.at[pl.ds(base, CPT)])
    pltpu.sync_copy(whp_v, whp_hbm.at[pl.ds(base, CPT)])
    pltpu.sync_copy(dlp_v, dlp_hbm.at[pl.ds(wid * NP, NP)])
    pltpu.sync_copy(dhp_v, dhp_hbm.at[pl.ds(wid * NP, NP)])


def _k3_body(dlp_ref, dhp_ref, dislp_ref, dishp_ref):
    acc_lp = dlp_ref[pl.ds(0, NP)]
    acc_hp = dhp_ref[pl.ds(0, NP)]
    for i in range(1, NW):
        acc_lp = acc_lp + dlp_ref[pl.ds(i * NP, NP)]
        acc_hp = acc_hp + dhp_ref[pl.ds(i * NP, NP)]
    dislp_ref[...] = lax.rsqrt(acc_lp + (1.0 + EOS) + EOS)
    dishp_ref[...] = lax.rsqrt(acc_hp + (1.0 + EOS) + EOS)


def _run_k3(dlp_part, dhp_part):
    return pl.pallas_call(
        _k3_body,
        out_shape=[
            jax.ShapeDtypeStruct((NP,), jnp.float32),
            jax.ShapeDtypeStruct((NP,), jnp.float32),
        ],
    )(dlp_part, dhp_part)


@functools.partial(
    pl.kernel,
    out_type=[
        jax.ShapeDtypeStruct((E + N,), jnp.float32),
        jax.ShapeDtypeStruct((E + N,), jnp.float32),
    ],
    mesh=_MESH,
    compiler_params=_SC_PARAMS,
    scratch_types=[
        pltpu.VMEM((N,), jnp.float32),
        pltpu.VMEM((N,), jnp.float32),
        pltpu.VMEM((CPT,), jnp.int32),
        pltpu.VMEM((CPT,), jnp.int32),
        pltpu.VMEM((CPT,), jnp.float32),
        pltpu.VMEM((CPT,), jnp.float32),
        pltpu.VMEM((CPT,), jnp.float32),
        pltpu.VMEM((TAIL_CPT,), jnp.float32),
        pltpu.VMEM((TAIL_CPT,), jnp.float32),
    ],
)
def _k4(dislp_hbm, dishp_hbm, src_hbm, dst_hbm, wlp_hbm,
        olp_hbm, ohp_hbm,
        dislp_v, dishp_v, src_v, dst_v, wlp_v, olp_v, ohp_v, tlp_v, thp_v):
    wid = lax.axis_index("s") * NC + lax.axis_index("c")
    base = wid * CPT
    pltpu.sync_copy(dislp_hbm.at[pl.ds(0, N)], dislp_v)
    pltpu.sync_copy(dishp_hbm.at[pl.ds(0, N)], dishp_v)
    pltpu.sync_copy(src_hbm.at[pl.ds(base, CPT)], src_v)
    pltpu.sync_copy(dst_hbm.at[pl.ds(base, CPT)], dst_v)
    pltpu.sync_copy(wlp_hbm.at[pl.ds(base, CPT)], wlp_v)

    @plsc.parallel_loop(0, CPT // L, 1, unroll=5)
    def _edge(i):
        o = i * L
        s = src_v[pl.ds(o, L)]
        d = dst_v[pl.ds(o, L)]
        w = wlp_v[pl.ds(o, L)]
        a = plsc.load_gather(dislp_v, [s]) * plsc.load_gather(dislp_v, [d])
        b = plsc.load_gather(dishp_v, [s]) * plsc.load_gather(dishp_v, [d])
        olp_v[pl.ds(o, L)] = (w + EOS) * a
        ohp_v[pl.ds(o, L)] = -((1.0 - w) + EOS) * b

    pltpu.sync_copy(olp_v, olp_hbm.at[pl.ds(base, CPT)])
    pltpu.sync_copy(ohp_v, ohp_hbm.at[pl.ds(base, CPT)])

    @pl.when(wid < TAIL_TILES)
    def _tail():
        tb = wid * TAIL_CPT
        ones = jnp.ones((L,), jnp.float32)

        @plsc.parallel_loop(0, TAIL_CPT // L, 1, unroll=5)
        def _t(k):
            dl = dislp_v[pl.ds(tb + k * L, L)]
            tlp_v[pl.ds(k * L, L)] = (1.0 + EOS) * dl * dl
            thp_v[pl.ds(k * L, L)] = ones

        pltpu.sync_copy(tlp_v, olp_hbm.at[pl.ds(E + tb, TAIL_CPT)])
        pltpu.sync_copy(thp_v, ohp_hbm.at[pl.ds(E + tb, TAIL_CPT)])


def kernel(features, edges, W_emb, b_emb, W_edge, b_edge):
    b_emb2 = b_emb.reshape(1, H)
    v2 = (0.5 * (W_edge[:H, 0] + W_edge[H:, 0])).reshape(1, H)
    be2 = (0.5 * b_edge).reshape(1, 1)
    g = jnp.asarray(_G_CONST)

    p, src, dst = _run_k1(features, W_emb, b_emb2, v2, be2, edges)

    wlp, whp, dlp_part, dhp_part = _k2(p, src, dst, g)

    dis_lp, dis_hp = _run_k3(dlp_part, dhp_part)

    w_lp_norm, w_hp_norm = _k4(dis_lp, dis_hp, src, dst, wlp)

    loop = jnp.arange(N, dtype=edges.dtype)
    edges_full = jnp.concatenate([edges, jnp.stack([loop, loop])], axis=1)
    return (edges_full, w_lp_norm, edges_full, w_hp_norm, wlp, whp)


# speedup vs baseline: 2.0406x; 1.1210x over previous
import functools

import jax
import jax.numpy as jnp
import numpy as np
from jax import lax
from jax.experimental import pallas as pl
from jax.experimental.pallas import tpu as pltpu
from jax.experimental.pallas import tpu_sc as plsc

N = 10000
E = 320000
D = 128
H = 128
NP = 10240
EOS = 1e-10
ALPHA = 1.0
TEMP = 1.0
BIAS = 0.0001
NC = 2
NS = 16
NW = NC * NS
CPT = E // NW
L = 16
TAIL_TILES = 25
TAIL_CPT = N // TAIL_TILES


def _gumbel_logit():
    def threefry2x32(k1, k2, x0, x1):
        rotations = [np.array([13, 15, 26, 6]), np.array([17, 29, 16, 24])]
        ks = [np.uint32(k1), np.uint32(k2),
              np.uint32(k1) ^ np.uint32(k2) ^ np.uint32(0x1BD11BDA)]
        x = [(x0 + ks[0]).astype(np.uint32), (x1 + ks[1]).astype(np.uint32)]
        for i in range(5):
            for r in rotations[i % 2]:
                x[0] = (x[0] + x[1]).astype(np.uint32)
                x[1] = ((x[1] << np.uint32(r))
                        | (x[1] >> np.uint32(32 - r))).astype(np.uint32)
                x[1] = x[1] ^ x[0]
            x[0] = (x[0] + ks[(i + 1) % 3]).astype(np.uint32)
            x[1] = (x[1] + ks[(i + 2) % 3] + np.uint32(i + 1)).astype(np.uint32)
        return x

    b1, b2 = threefry2x32(0, 42, np.zeros((E,), np.uint32),
                          np.arange(E, dtype=np.uint32))
    bits = b1 ^ b2
    u = ((bits >> np.uint32(9)) | np.uint32(0x3F800000)).view(np.float32) \
        - np.float32(1.0)
    u = np.maximum(np.float32(0.0), u)
    eps = (np.float32(BIAS - (1.0 - BIAS)) * u + np.float32(1.0 - BIAS))
    return np.log(eps) - np.log(np.float32(1.0) - eps)


_G_CONST = _gumbel_logit()


def _k1_body(x_ref, w_ref, b_ref, v_ref, be_ref, e_ref,
             p_ref, src_ref, dst_ref):
    h = jax.nn.relu(
        jnp.dot(x_ref[...], w_ref[...], preferred_element_type=jnp.float32)
        + b_ref[...])
    p = lax.dot_general(
        v_ref[...], h, (((1,), (1,)), ((), ())),
        preferred_element_type=jnp.float32) + be_ref[...]
    p_ref[...] = p.reshape(N)
    e = e_ref[...]
    src_ref[...] = e[0]
    dst_ref[...] = e[1]


def _run_k1(x, w_emb, b_emb2, v2, be2, edges):
    return pl.pallas_call(
        _k1_body,
        out_shape=[
            jax.ShapeDtypeStruct((N,), jnp.float32),
            jax.ShapeDtypeStruct((E,), jnp.int32),
            jax.ShapeDtypeStruct((E,), jnp.int32),
        ],
    )(x, w_emb, b_emb2, v2, be2, edges)


_MESH = plsc.VectorSubcoreMesh(
    core_axis_name="c", subcore_axis_name="s", num_cores=NC, num_subcores=NS)
_SC_PARAMS = pltpu.CompilerParams(needs_layout_passes=False)


@functools.partial(
    pl.kernel,
    out_type=[
        jax.ShapeDtypeStruct((E,), jnp.float32),
        jax.ShapeDtypeStruct((E,), jnp.float32),
        jax.ShapeDtypeStruct((NW * NP,), jnp.float32),
        jax.ShapeDtypeStruct((NW * NP,), jnp.float32),
    ],
    mesh=_MESH,
    compiler_params=_SC_PARAMS,
    scratch_types=[
        pltpu.VMEM((N,), jnp.float32),
        pltpu.VMEM((CPT,), jnp.int32),
        pltpu.VMEM((CPT,), jnp.int32),
        pltpu.VMEM((CPT,), jnp.float32),
        pltpu.VMEM((CPT,), jnp.float32),
        pltpu.VMEM((CPT,), jnp.float32),
        pltpu.VMEM((NP,), jnp.float32),
        pltpu.VMEM((NP,), jnp.float32),
        pltpu.SemaphoreType.DMA,
    ],
)
def _k2(p_hbm, src_hbm, dst_hbm, g_hbm,
        wlp_hbm, whp_hbm, dlp_hbm, dhp_hbm,
        p_v, src_v, dst_v, g_v, wlp_v, whp_v, dlp_v, dhp_v, sem):
    wid = lax.axis_index("s") * NC + lax.axis_index("c")
    base = wid * CPT
    cps = [
        pltpu.async_copy(p_hbm.at[pl.ds(0, N)], p_v, sem),
        pltpu.async_copy(src_hbm.at[pl.ds(base, CPT)], src_v, sem),
        pltpu.async_copy(dst_hbm.at[pl.ds(base, CPT)], dst_v, sem),
        pltpu.async_copy(g_hbm.at[pl.ds(base, CPT)], g_v, sem),
    ]

    zero = jnp.zeros((L,), jnp.float32)

    @plsc.parallel_loop(0, NP // L, 1, unroll=8)
    def _zero(i):
        dlp_v[pl.ds(i * L, L)] = zero
        dhp_v[pl.ds(i * L, L)] = zero

    for c in cps:
        c.wait()

    @plsc.parallel_loop(0, CPT // L, 1, unroll=5)
    def _edge(i):
        o = i * L
        s = src_v[pl.ds(o, L)]
        d = dst_v[pl.ds(o, L)]
        gg = g_v[pl.ds(o, L)]
        ps = plsc.load_gather(p_v, [s])
        pd = plsc.load_gather(p_v, [d])
        gate = gg + ps + pd
        wlp = 1.0 / (1.0 + jnp.exp(-gate))
        whp = 1.0 - wlp
        wlp_v[pl.ds(o, L)] = wlp
        whp_v[pl.ds(o, L)] = whp
        plsc.addupdate_scatter(dlp_v, [s], wlp + EOS)
        plsc.addupdate_scatter(dhp_v, [s], whp + EOS)

    outs = [
        pltpu.async_copy(wlp_v, wlp_hbm.at[pl.ds(base, CPT)], sem),
        pltpu.async_copy(whp_v, whp_hbm.at[pl.ds(base, CPT)], sem),
        pltpu.async_copy(dlp_v, dlp_hbm.at[pl.ds(wid * NP, NP)], sem),
        pltpu.async_copy(dhp_v, dhp_hbm.at[pl.ds(wid * NP, NP)], sem),
    ]
    for c in outs:
        c.wait()


def _k3_body(dlp_ref, dhp_ref, dislp_ref, dishp_ref):
    acc_lp = dlp_ref[pl.ds(0, NP)]
    acc_hp = dhp_ref[pl.ds(0, NP)]
    for i in range(1, NW):
        acc_lp = acc_lp + dlp_ref[pl.ds(i * NP, NP)]
        acc_hp = acc_hp + dhp_ref[pl.ds(i * NP, NP)]
    dislp_ref[...] = lax.rsqrt(acc_lp + (1.0 + EOS) + EOS)
    dishp_ref[...] = lax.rsqrt(acc_hp + (1.0 + EOS) + EOS)


def _run_k3(dlp_part, dhp_part):
    return pl.pallas_call(
        _k3_body,
        out_shape=[
            jax.ShapeDtypeStruct((NP,), jnp.float32),
            jax.ShapeDtypeStruct((NP,), jnp.float32),
        ],
    )(dlp_part, dhp_part)


@functools.partial(
    pl.kernel,
    out_type=[
        jax.ShapeDtypeStruct((E + N,), jnp.float32),
        jax.ShapeDtypeStruct((E + N,), jnp.float32),
    ],
    mesh=_MESH,
    compiler_params=_SC_PARAMS,
    scratch_types=[
        pltpu.VMEM((N,), jnp.float32),
        pltpu.VMEM((N,), jnp.float32),
        pltpu.VMEM((CPT,), jnp.int32),
        pltpu.VMEM((CPT,), jnp.int32),
        pltpu.VMEM((CPT,), jnp.float32),
        pltpu.VMEM((CPT,), jnp.float32),
        pltpu.VMEM((CPT,), jnp.float32),
        pltpu.VMEM((TAIL_CPT,), jnp.float32),
        pltpu.VMEM((TAIL_CPT,), jnp.float32),
        pltpu.SemaphoreType.DMA,
    ],
)
def _k4(dislp_hbm, dishp_hbm, src_hbm, dst_hbm, wlp_hbm,
        olp_hbm, ohp_hbm,
        dislp_v, dishp_v, src_v, dst_v, wlp_v, olp_v, ohp_v, tlp_v, thp_v,
        sem):
    wid = lax.axis_index("s") * NC + lax.axis_index("c")
    base = wid * CPT
    cps = [
        pltpu.async_copy(dislp_hbm.at[pl.ds(0, N)], dislp_v, sem),
        pltpu.async_copy(dishp_hbm.at[pl.ds(0, N)], dishp_v, sem),
        pltpu.async_copy(src_hbm.at[pl.ds(base, CPT)], src_v, sem),
        pltpu.async_copy(dst_hbm.at[pl.ds(base, CPT)], dst_v, sem),
        pltpu.async_copy(wlp_hbm.at[pl.ds(base, CPT)], wlp_v, sem),
    ]
    for c in cps:
        c.wait()

    @plsc.parallel_loop(0, CPT // L, 1, unroll=5)
    def _edge(i):
        o = i * L
        s = src_v[pl.ds(o, L)]
        d = dst_v[pl.ds(o, L)]
        w = wlp_v[pl.ds(o, L)]
        a = plsc.load_gather(dislp_v, [s]) * plsc.load_gather(dislp_v, [d])
        b = plsc.load_gather(dishp_v, [s]) * plsc.load_gather(dishp_v, [d])
        olp_v[pl.ds(o, L)] = (w + EOS) * a
        ohp_v[pl.ds(o, L)] = -((1.0 - w) + EOS) * b

    outs = [
        pltpu.async_copy(olp_v, olp_hbm.at[pl.ds(base, CPT)], sem),
        pltpu.async_copy(ohp_v, ohp_hbm.at[pl.ds(base, CPT)], sem),
    ]

    @pl.when(wid < TAIL_TILES)
    def _tail():
        tb = wid * TAIL_CPT
        ones = jnp.ones((L,), jnp.float32)

        @plsc.parallel_loop(0, TAIL_CPT // L, 1, unroll=5)
        def _t(k):
            dl = dislp_v[pl.ds(tb + k * L, L)]
            tlp_v[pl.ds(k * L, L)] = (1.0 + EOS) * dl * dl
            thp_v[pl.ds(k * L, L)] = ones

        pltpu.sync_copy(tlp_v, olp_hbm.at[pl.ds(E + tb, TAIL_CPT)])
        pltpu.sync_copy(thp_v, ohp_hbm.at[pl.ds(E + tb, TAIL_CPT)])

    for c in outs:
        c.wait()


def kernel(features, edges, W_emb, b_emb, W_edge, b_edge):
    b_emb2 = b_emb.reshape(1, H)
    v2 = (0.5 * (W_edge[:H, 0] + W_edge[H:, 0])).reshape(1, H)
    be2 = (0.5 * b_edge).reshape(1, 1)
    g = jnp.asarray(_G_CONST)

    p, src, dst = _run_k1(features, W_emb, b_emb2, v2, be2, edges)

    wlp, whp, dlp_part, dhp_part = _k2(p, src, dst, g)

    dis_lp, dis_hp = _run_k3(dlp_part, dhp_part)

    w_lp_norm, w_hp_norm = _k4(dis_lp, dis_hp, src, dst, wlp)

    loop = jnp.arange(N, dtype=edges.dtype)
    edges_full = jnp.concatenate([edges, jnp.stack([loop, loop])], axis=1)
    return (edges_full, w_lp_norm, edges_full, w_hp_norm, wlp, whp)
